# Initial kernel scaffold; baseline (speedup 1.0000x reference)
import jax
import jax.numpy as jnp
from jax.experimental import pallas as pl


def kernel(mem, buffer_label, idx_keys, idx_vals, x, y):
    raise NotImplementedError("write your pallas kernel here")


# speedup vs baseline: 9.4701x; 9.4701x over previous
import functools

import jax
import jax.numpy as jnp
from jax import lax
from jax.experimental import pallas as pl
from jax.experimental.pallas import tpu as pltpu
from jax.experimental.pallas import tpu_sc as plsc

_M = 10000
_B = 4096
_IMG = (3, 32, 32)
_ROW = 3 * 32 * 32

_NC = 2
_NS = 16
_NW = _NC * _NS
_BPW = _B // _NW
_L = 16
_VPW = _BPW // _L


def _mask_body(keys_ref, mask_ref, pos_ref, cnt_ref):
  pid = pl.program_id(0)
  kb = keys_ref[0, pl.ds(pid * _BPW, _BPW)].reshape(_BPW, 1)
  kall = keys_ref[...].reshape(1, _B)
  i_col = pid * _BPW + lax.broadcasted_iota(jnp.int32, (_BPW, 1), 0)
  j_row = lax.broadcasted_iota(jnp.int32, (1, _B), 1)
  dup = jnp.where((kb == kall) & (j_row > i_col), 1, 0)
  m = 1 - jnp.max(dup, axis=1)
  mask_ref[0, pl.ds(pid * _BPW, _BPW)] = m

  mf = m.astype(jnp.float32).reshape(1, _BPW)
  jj = lax.broadcasted_iota(jnp.int32, (_BPW, _BPW), 0)
  ii = lax.broadcasted_iota(jnp.int32, (_BPW, _BPW), 1)
  lt = jnp.where(jj < ii, 1.0, 0.0)
  pos = jnp.dot(mf, lt, preferred_element_type=jnp.float32)
  pos_ref[0, pl.ds(pid * _BPW, _BPW)] = pos.reshape(_BPW).astype(jnp.int32)

  cnt = jnp.sum(m)
  cnt_ref[0, pl.ds(pid * _BPW, _BPW)] = jnp.full((_BPW,), cnt, jnp.int32)


_mask_call = pl.pallas_call(
    _mask_body,
    grid=(_NW,),
    in_specs=[pl.BlockSpec((1, _B), lambda i: (0, 0))],
    out_specs=[pl.BlockSpec((1, _B), lambda i: (0, 0))] * 3,
    out_shape=[jax.ShapeDtypeStruct((1, _B), jnp.int32)] * 3,
)


def _sc_body(mem_hbm, lab_hbm, keys_hbm, vals_hbm, x_hbm, y_hbm, mask_hbm,
             pos_hbm, cnt_hbm,
             keys_v, vals_v, mask_v, pos_v, cnt_v, ck, cv, stage_k, stage_v,
             rows, lab_all, y_all, keys_all, vals_all, mask_all, gsem, ssem):
  wid = lax.axis_index("s") * _NC + lax.axis_index("c")
  base = wid * _BPW

  pltpu.sync_copy(keys_hbm.at[pl.ds(base, _BPW)], keys_v)
  pltpu.sync_copy(vals_hbm.at[pl.ds(base, _BPW)], vals_v)
  pltpu.sync_copy(mask_hbm.at[pl.ds(base, _BPW)], mask_v)
  pltpu.sync_copy(pos_hbm.at[pl.ds(base, _BPW)], pos_v)
  pltpu.sync_copy(cnt_hbm.at[pl.ds(base, _L)], cnt_v)

  n = cnt_v[...][0]
  nch = (n + _L - 1) // _L

  def compact(t, c):
    kv = keys_v[pl.ds(t * _L, _L)]
    vv = vals_v[pl.ds(t * _L, _L)]
    mv = mask_v[pl.ds(t * _L, _L)] > 0
    pos = pos_v[pl.ds(t * _L, _L)]
    plsc.store_scatter(ck, [pos], kv, mask=mv)
    plsc.store_scatter(cv, [pos], vv, mask=mv)
    return c

  lax.fori_loop(0, _VPW, compact, 0)

  @pl.when(n > 0)
  def _():
    k0 = jnp.broadcast_to(ck[pl.ds(0, _L)][0], (_L,))
    v0 = jnp.broadcast_to(cv[pl.ds(0, _L)][0], (_L,))
    lane = lax.iota(jnp.int32, _L)
    tail = (nch - 1) * _L
    rem = n - tail
    kt = ck[pl.ds(tail, _L)]
    vt = cv[pl.ds(tail, _L)]
    ck[pl.ds(tail, _L)] = jnp.where(lane < rem, kt, k0)
    cv[pl.ds(tail, _L)] = jnp.where(lane < rem, vt, v0)

  @pl.when(wid == 0)
  def _():
    pltpu.sync_copy(y_hbm, y_all)
    pltpu.sync_copy(keys_hbm, keys_all)
    pltpu.sync_copy(vals_hbm, vals_all)
    pltpu.sync_copy(mask_hbm, mask_all)
    pltpu.sync_copy(lab_hbm, lab_all)

    def lbody(t, c):
      kv = keys_all[pl.ds(t * _L, _L)]
      vv = vals_all[pl.ds(t * _L, _L)]
      mv = mask_all[pl.ds(t * _L, _L)] > 0
      yv = plsc.load_gather(y_all, [vv])
      plsc.store_scatter(lab_all, [kv], yv, mask=mv)
      return c

    lax.fori_loop(0, _B // _L, lbody, 0)
    pltpu.sync_copy(lab_all, lab_hbm)

  def rbody(c, carry):
    stage_v[...] = cv[pl.ds(c * _L, _L)]
    stage_k[...] = ck[pl.ds(c * _L, _L)]
    pltpu.async_copy(x_hbm.at[stage_v], rows, gsem).wait()
    pltpu.async_copy(rows, mem_hbm.at[stage_k], ssem).wait()
    return carry

  lax.fori_loop(0, nch, rbody, 0)


@functools.cache
def _get_sc_call():
  return functools.partial(
      pl.kernel,
      out_type=(),
      mesh=plsc.VectorSubcoreMesh(core_axis_name="c", subcore_axis_name="s"),
      compiler_params=pltpu.CompilerParams(needs_layout_passes=False),
      scratch_types=[
          pltpu.VMEM((_BPW,), jnp.int32),
          pltpu.VMEM((_BPW,), jnp.int32),
          pltpu.VMEM((_BPW,), jnp.int32),
          pltpu.VMEM((_BPW,), jnp.int32),
          pltpu.VMEM((_L,), jnp.int32),
          pltpu.VMEM((_BPW,), jnp.int32),
          pltpu.VMEM((_BPW,), jnp.int32),
          pltpu.VMEM((_L,), jnp.int32),
          pltpu.VMEM((_L,), jnp.int32),
          pltpu.VMEM((_L, _ROW), jnp.float32),
          pltpu.VMEM((_M,), jnp.int32),
          pltpu.VMEM((_B,), jnp.int32),
          pltpu.VMEM((_B,), jnp.int32),
          pltpu.VMEM((_B,), jnp.int32),
          pltpu.VMEM((_B,), jnp.int32),
          pltpu.SemaphoreType.DMA,
          pltpu.SemaphoreType.DMA,
      ],
  )(_sc_body)


def kernel(mem, buffer_label, idx_keys, idx_vals, x, y):
  mask, pos, cnt = _mask_call(idx_keys.reshape(1, _B))
  mem_ref = jax.new_ref(mem.reshape(_M, _ROW))
  lab_ref = jax.new_ref(buffer_label)
  _get_sc_call()(mem_ref, lab_ref, idx_keys, idx_vals, x.reshape(_B, _ROW), y,
                 mask.reshape(_B), pos.reshape(_B), cnt.reshape(_B))
  return mem_ref[...].reshape((_M,) + _IMG), lab_ref[...]


# speedup vs baseline: 9.5947x; 1.0132x over previous
import functools

import jax
import jax.numpy as jnp
from jax import lax
from jax.experimental import pallas as pl
from jax.experimental.pallas import tpu as pltpu
from jax.experimental.pallas import tpu_sc as plsc

_M = 10000
_B = 4096
_IMG = (3, 32, 32)
_ROW = 3 * 32 * 32

_NC = 2
_NS = 16
_NW = _NC * _NS
_BPW = _B // _NW
_L = 16
_VPW = _BPW // _L


def _mask_body(keys_ref, mask_ref, pos_ref, cnt_ref):
  pid = pl.program_id(0)
  kb = keys_ref[0, pl.ds(pid * _BPW, _BPW)].reshape(_BPW, 1)
  kall = keys_ref[...].reshape(1, _B)
  i_col = pid * _BPW + lax.broadcasted_iota(jnp.int32, (_BPW, 1), 0)
  j_row = lax.broadcasted_iota(jnp.int32, (1, _B), 1)
  dup = jnp.where((kb == kall) & (j_row > i_col), 1, 0)
  m = 1 - jnp.max(dup, axis=1)
  mask_ref[0, pl.ds(pid * _BPW, _BPW)] = m

  mf = m.astype(jnp.float32).reshape(1, _BPW)
  jj = lax.broadcasted_iota(jnp.int32, (_BPW, _BPW), 0)
  ii = lax.broadcasted_iota(jnp.int32, (_BPW, _BPW), 1)
  lt = jnp.where(jj < ii, 1.0, 0.0)
  pos = jnp.dot(mf, lt, preferred_element_type=jnp.float32)
  pos_ref[0, pl.ds(pid * _BPW, _BPW)] = pos.reshape(_BPW).astype(jnp.int32)

  cnt = jnp.sum(m)
  cnt_ref[0, pl.ds(pid * _BPW, _BPW)] = jnp.full((_BPW,), cnt, jnp.int32)


_mask_call = pl.pallas_call(
    _mask_body,
    grid=(_NW,),
    in_specs=[pl.BlockSpec((1, _B), lambda i: (0, 0))],
    out_specs=[pl.BlockSpec((1, _B), lambda i: (0, 0))] * 3,
    out_shape=[jax.ShapeDtypeStruct((1, _B), jnp.int32)] * 3,
)


def _sc_body(mem_hbm, lab_hbm, keys_hbm, vals_hbm, x_hbm, y_hbm, mask_hbm,
             pos_hbm, cnt_hbm,
             keys_v, vals_v, mask_v, pos_v, cnt_v, ck, cv,
             stage_k0, stage_k1, stage_v0, stage_v1, rows0, rows1,
             lab_all, y_all, keys_all, vals_all, mask_all,
             gsem0, gsem1, ssem0, ssem1):
  stage_k = (stage_k0, stage_k1)
  stage_v = (stage_v0, stage_v1)
  rows = (rows0, rows1)
  gsem = (gsem0, gsem1)
  ssem = (ssem0, ssem1)
  wid = lax.axis_index("s") * _NC + lax.axis_index("c")
  base = wid * _BPW

  pltpu.sync_copy(keys_hbm.at[pl.ds(base, _BPW)], keys_v)
  pltpu.sync_copy(vals_hbm.at[pl.ds(base, _BPW)], vals_v)
  pltpu.sync_copy(mask_hbm.at[pl.ds(base, _BPW)], mask_v)
  pltpu.sync_copy(pos_hbm.at[pl.ds(base, _BPW)], pos_v)
  pltpu.sync_copy(cnt_hbm.at[pl.ds(base, _L)], cnt_v)

  n = cnt_v[...][0]
  nch = (n + _L - 1) // _L

  def compact(t, c):
    kv = keys_v[pl.ds(t * _L, _L)]
    vv = vals_v[pl.ds(t * _L, _L)]
    mv = mask_v[pl.ds(t * _L, _L)] > 0
    pos = pos_v[pl.ds(t * _L, _L)]
    plsc.store_scatter(ck, [pos], kv, mask=mv)
    plsc.store_scatter(cv, [pos], vv, mask=mv)
    return c

  lax.fori_loop(0, _VPW, compact, 0)

  @pl.when(n > 0)
  def _():
    k0 = jnp.broadcast_to(ck[pl.ds(0, _L)][0], (_L,))
    v0 = jnp.broadcast_to(cv[pl.ds(0, _L)][0], (_L,))
    lane = lax.iota(jnp.int32, _L)
    tail = (nch - 1) * _L
    rem = n - tail
    kt = ck[pl.ds(tail, _L)]
    vt = cv[pl.ds(tail, _L)]
    ck[pl.ds(tail, _L)] = jnp.where(lane < rem, kt, k0)
    cv[pl.ds(tail, _L)] = jnp.where(lane < rem, vt, v0)

  @pl.when(wid == 0)
  def _():
    pltpu.sync_copy(y_hbm, y_all)
    pltpu.sync_copy(keys_hbm, keys_all)
    pltpu.sync_copy(vals_hbm, vals_all)
    pltpu.sync_copy(mask_hbm, mask_all)
    pltpu.sync_copy(lab_hbm, lab_all)

    def lbody(t, c):
      kv = keys_all[pl.ds(t * _L, _L)]
      vv = vals_all[pl.ds(t * _L, _L)]
      mv = mask_all[pl.ds(t * _L, _L)] > 0
      yv = plsc.load_gather(y_all, [vv])
      plsc.store_scatter(lab_all, [kv], yv, mask=mv)
      return c

    lax.fori_loop(0, _B // _L, lbody, 0)
    pltpu.sync_copy(lab_all, lab_hbm)

  for c in range(_VPW + 1):
    if c < _VPW:
      b = c & 1

      @pl.when(c < nch)
      def _(c=c, b=b):
        if c >= 2:
          pltpu.make_async_copy(rows[b], mem_hbm.at[stage_k[b]],
                                ssem[b]).wait()
        stage_v[b][...] = cv[pl.ds(c * _L, _L)]
        stage_k[b][...] = ck[pl.ds(c * _L, _L)]
        pltpu.async_copy(x_hbm.at[stage_v[b]], rows[b], gsem[b])
    if c >= 1:
      cc = c - 1
      b = cc & 1

      @pl.when(cc < nch)
      def _(cc=cc, b=b):
        pltpu.make_async_copy(x_hbm.at[stage_v[b]], rows[b], gsem[b]).wait()
        pltpu.async_copy(rows[b], mem_hbm.at[stage_k[b]], ssem[b])

  @pl.when(nch >= 2)
  def _():
    pltpu.make_async_copy(rows[0], mem_hbm.at[stage_k[0]], ssem[0]).wait()
    pltpu.make_async_copy(rows[1], mem_hbm.at[stage_k[1]], ssem[1]).wait()

  @pl.when(nch == 1)
  def _():
    pltpu.make_async_copy(rows[0], mem_hbm.at[stage_k[0]], ssem[0]).wait()


@functools.cache
def _get_sc_call():
  return functools.partial(
      pl.kernel,
      out_type=(),
      mesh=plsc.VectorSubcoreMesh(core_axis_name="c", subcore_axis_name="s"),
      compiler_params=pltpu.CompilerParams(needs_layout_passes=False),
      scratch_types=[
          pltpu.VMEM((_BPW,), jnp.int32),
          pltpu.VMEM((_BPW,), jnp.int32),
          pltpu.VMEM((_BPW,), jnp.int32),
          pltpu.VMEM((_BPW,), jnp.int32),
          pltpu.VMEM((_L,), jnp.int32),
          pltpu.VMEM((_BPW,), jnp.int32),
          pltpu.VMEM((_BPW,), jnp.int32),
          pltpu.VMEM((_L,), jnp.int32),
          pltpu.VMEM((_L,), jnp.int32),
          pltpu.VMEM((_L,), jnp.int32),
          pltpu.VMEM((_L,), jnp.int32),
          pltpu.VMEM((_L, _ROW), jnp.float32),
          pltpu.VMEM((_L, _ROW), jnp.float32),
          pltpu.VMEM((_M,), jnp.int32),
          pltpu.VMEM((_B,), jnp.int32),
          pltpu.VMEM((_B,), jnp.int32),
          pltpu.VMEM((_B,), jnp.int32),
          pltpu.VMEM((_B,), jnp.int32),
          pltpu.SemaphoreType.DMA,
          pltpu.SemaphoreType.DMA,
          pltpu.SemaphoreType.DMA,
          pltpu.SemaphoreType.DMA,
      ],
  )(_sc_body)


def kernel(mem, buffer_label, idx_keys, idx_vals, x, y):
  mask, pos, cnt = _mask_call(idx_keys.reshape(1, _B))
  mem_ref = jax.new_ref(mem.reshape(_M, _ROW))
  lab_ref = jax.new_ref(buffer_label)
  _get_sc_call()(mem_ref, lab_ref, idx_keys, idx_vals, x.reshape(_B, _ROW), y,
                 mask.reshape(_B), pos.reshape(_B), cnt.reshape(_B))
  return mem_ref[...].reshape((_M,) + _IMG), lab_ref[...]
